# Initial kernel scaffold; baseline (speedup 1.0000x reference)
import jax
import jax.numpy as jnp
from jax.experimental import pallas as pl


def kernel(x, edge_index, W1, b1, W2, b2, Wlin, blin):
    raise NotImplementedError("write your pallas kernel here")


# speedup vs baseline: 2.2055x; 2.2055x over previous
import functools

import jax
import jax.numpy as jnp
from jax import lax
from jax.experimental import pallas as pl
from jax.experimental.pallas import tpu as pltpu
from jax.experimental.pallas import tpu_sc as plsc

N = 10000
NPAD = 10240
E = 320000
CHUNK = 128
NCHUNKS = 2560
EPAD = NCHUNKS * CHUNK
D_IN = 128
H = 256
HH = 128
D_OUT = 128
DEGW = 16

NSC = 2
NTILES = 16
NPASS = 4
PASS_ROWS = NPAD // NPASS
ACC_ROWS = 3072
DUMP = PASS_ROWS
ROWS_PER_TILE = NPAD // NTILES
WB_PER_TILE = PASS_ROWS // NTILES
ZERO_PER_TILE = ACC_ROWS // NTILES
ZERO_STEPS = (128, 64)
DEG_CHUNKS_PER_TILE = NCHUNKS // (NSC * NTILES)
PROP_CHUNKS_PER_TILE = NCHUNKS // NTILES

BR = 512
GRID = NPAD // BR

_mesh = plsc.VectorSubcoreMesh(core_axis_name="c", subcore_axis_name="s")


def _zero_fill(buf, nrows, width):
    zv = jnp.zeros((16,), jnp.float32)

    def row(i, _):
        for j in range(width // 16):
            buf[i, pl.ds(j * 16, 16)] = zv
        return 0

    lax.fori_loop(0, nrows, row, 0)


@functools.partial(
    pl.kernel,
    mesh=_mesh,
    out_type=jax.ShapeDtypeStruct((NSC, NPAD, DEGW), jnp.float32),
    scratch_types=[
        pltpu.VMEM((DEG_CHUNKS_PER_TILE, CHUNK), jnp.int32),
        pltpu.VMEM((CHUNK, DEGW), jnp.float32),
        pltpu.VMEM((CHUNK, DEGW), jnp.float32),
        pltpu.VMEM_SHARED((NPAD, DEGW), jnp.float32),
    ],
)
def _deg_kernel(dst_hbm, out_hbm, idx_v, ones_v, zbuf, acc_sh):
    c = lax.axis_index("c")
    s = lax.axis_index("s")
    wid = c * NTILES + s

    ones = jnp.full((16,), 1.0, jnp.float32)

    def fill(i, _):
        ones_v[i, :] = ones
        zbuf[i, :] = jnp.zeros((16,), jnp.float32)
        return 0

    lax.fori_loop(0, CHUNK, fill, 0)

    for k in range(ROWS_PER_TILE // CHUNK):
        pltpu.sync_copy(zbuf, acc_sh.at[pl.ds(s * ROWS_PER_TILE + k * CHUNK, CHUNK)])
    plsc.subcore_barrier()

    pltpu.sync_copy(dst_hbm.at[pl.ds(wid * DEG_CHUNKS_PER_TILE, DEG_CHUNKS_PER_TILE)], idx_v)

    def body(cc, _):
        pltpu.sync_copy(ones_v, acc_sh.at[idx_v.at[cc]], add=True)
        return 0

    lax.fori_loop(0, DEG_CHUNKS_PER_TILE, body, 0)
    plsc.subcore_barrier()
    pltpu.sync_copy(
        acc_sh.at[pl.ds(s * ROWS_PER_TILE, ROWS_PER_TILE)],
        out_hbm.at[c, pl.ds(s * ROWS_PER_TILE, ROWS_PER_TILE)],
    )


@functools.partial(
    pl.kernel,
    mesh=_mesh,
    out_type=jax.ShapeDtypeStruct((NSC, NPAD, HH), jnp.float32),
    scratch_types=[
        pltpu.VMEM((PROP_CHUNKS_PER_TILE, CHUNK), jnp.int32),
        pltpu.VMEM((PROP_CHUNKS_PER_TILE, CHUNK), jnp.int32),
        pltpu.VMEM((1, CHUNK), jnp.int32),
        pltpu.VMEM((CHUNK, HH), jnp.float32),
        pltpu.VMEM((CHUNK, HH), jnp.float32),
        pltpu.VMEM((CHUNK, HH), jnp.float32),
        pltpu.SemaphoreType.DMA,
        pltpu.SemaphoreType.DMA,
        pltpu.VMEM_SHARED((ACC_ROWS, HH), jnp.float32),
    ],
)
def _prop_kernel(g_hbm, src_hbm, dst_hbm, out_hbm,
                 srcv, dstv, rmap, rows_a, rows_b, zrow, sem_a, sem_b, acc_sh):
    c = lax.axis_index("c")
    s = lax.axis_index("s")

    _zero_fill(zrow, CHUNK, HH)

    base = s * PROP_CHUNKS_PER_TILE
    pltpu.sync_copy(src_hbm.at[pl.ds(base, PROP_CHUNKS_PER_TILE)], srcv)
    pltpu.sync_copy(dst_hbm.at[pl.ds(base, PROP_CHUNKS_PER_TILE)], dstv)

    gtab = g_hbm.at[c]

    for p in range(NPASS):
        row0 = jnp.int32(p * PASS_ROWS)

        zoff = 0
        for zn in ZERO_STEPS:
            pltpu.sync_copy(zrow.at[pl.ds(0, zn)],
                            acc_sh.at[pl.ds(s * ZERO_PER_TILE + zoff, zn)])
            zoff += zn
        plsc.subcore_barrier()

        def scatter(cc, buf):
            for j in range(CHUNK // 16):
                v = dstv[cc, pl.ds(j * 16, 16)]
                loc = v - row0
                ok = (loc >= 0) & (loc < PASS_ROWS)
                rmap[0, pl.ds(j * 16, 16)] = jnp.where(ok, loc, DUMP)
            pltpu.sync_copy(buf, acc_sh.at[rmap.at[0]], add=True)

        pltpu.async_copy(gtab.at[srcv.at[0]], rows_a, sem_a)
        pltpu.async_copy(gtab.at[srcv.at[1]], rows_b, sem_b)

        def body(g, _):
            c0 = 2 * g
            pltpu.make_async_copy(gtab.at[srcv.at[c0]], rows_a, sem_a).wait()
            scatter(c0, rows_a)
            pltpu.async_copy(gtab.at[srcv.at[c0 + 2]], rows_a, sem_a)
            pltpu.make_async_copy(gtab.at[srcv.at[c0 + 1]], rows_b, sem_b).wait()
            scatter(c0 + 1, rows_b)
            pltpu.async_copy(gtab.at[srcv.at[c0 + 3]], rows_b, sem_b)
            return 0

        lax.fori_loop(0, PROP_CHUNKS_PER_TILE // 2 - 1, body, 0)

        last = PROP_CHUNKS_PER_TILE - 2
        pltpu.make_async_copy(gtab.at[srcv.at[last]], rows_a, sem_a).wait()
        scatter(last, rows_a)
        pltpu.make_async_copy(gtab.at[srcv.at[last + 1]], rows_b, sem_b).wait()
        scatter(last + 1, rows_b)

        plsc.subcore_barrier()
        wb0 = s * WB_PER_TILE
        pltpu.sync_copy(
            acc_sh.at[pl.ds(wb0, WB_PER_TILE)],
            out_hbm.at[c, pl.ds(p * PASS_ROWS + wb0, WB_PER_TILE)],
        )
        plsc.subcore_barrier()


def _dinv_body(deg_ref, dinv_ref):
    deg = deg_ref[0, :, 0:1] + deg_ref[1, :, 0:1] + 1.0
    dinv_ref[...] = jnp.broadcast_to(lax.rsqrt(deg), (BR, 128))


def _pre_body(h_ref, w_ref, dinv_ref, g_ref):
    p = jnp.dot(h_ref[...], w_ref[...], preferred_element_type=jnp.float32)
    g = p * dinv_ref[:, 0:1]
    g_ref[0] = g[:, :HH]
    g_ref[1] = g[:, HH:]


def _post_body(m_ref, g_ref, dinv_ref, b_ref, h_ref):
    dinv = dinv_ref[:, 0:1]
    h0 = (m_ref[0] + g_ref[0]) * dinv
    h1 = (m_ref[1] + g_ref[1]) * dinv
    h_ref[...] = jnp.maximum(jnp.concatenate([h0, h1], axis=1) + b_ref[...], 0.0)


def _final_body(x_ref, h1_ref, h2_ref, wl_ref, bl_ref, out_ref):
    agg = jnp.concatenate([x_ref[...], h1_ref[...], h2_ref[...]], axis=1)
    out_ref[...] = jnp.dot(agg, wl_ref[...], preferred_element_type=jnp.float32) + bl_ref[...]


def _row_spec(width):
    return pl.BlockSpec((BR, width), lambda i: (i, 0))


def _half_spec(width):
    return pl.BlockSpec((NSC, BR, width), lambda i: (0, i, 0))


def _full_spec(shape):
    nd = len(shape)
    return pl.BlockSpec(shape, lambda i: (0,) * nd)


_dinv_kernel = pl.pallas_call(
    _dinv_body,
    grid=(GRID,),
    in_specs=[_half_spec(DEGW)],
    out_specs=_row_spec(128),
    out_shape=jax.ShapeDtypeStruct((NPAD, 128), jnp.float32),
)

_pre_kernel = pl.pallas_call(
    _pre_body,
    grid=(GRID,),
    in_specs=[_row_spec(H), _full_spec((H, H)), _row_spec(128)],
    out_specs=_half_spec(HH),
    out_shape=jax.ShapeDtypeStruct((NSC, NPAD, HH), jnp.float32),
)

_post_kernel = pl.pallas_call(
    _post_body,
    grid=(GRID,),
    in_specs=[_half_spec(HH), _half_spec(HH), _row_spec(128), _full_spec((1, H))],
    out_specs=_row_spec(H),
    out_shape=jax.ShapeDtypeStruct((NPAD, H), jnp.float32),
)

_final_kernel = pl.pallas_call(
    _final_body,
    grid=(GRID,),
    in_specs=[_row_spec(D_IN), _row_spec(H), _row_spec(H),
              _full_spec((D_IN + H + H, D_OUT)), _full_spec((1, D_OUT))],
    out_specs=_row_spec(D_OUT),
    out_shape=jax.ShapeDtypeStruct((NPAD, D_OUT), jnp.float32),
)


def kernel(x, edge_index, W1, b1, W2, b2, Wlin, blin):
    src = edge_index[0]
    dst = edge_index[1]
    pad = jnp.full((EPAD - E,), N, jnp.int32)
    srcp = jnp.concatenate([src, pad]).reshape(NCHUNKS, CHUNK)
    dstp = jnp.concatenate([dst, pad]).reshape(NCHUNKS, CHUNK)
    xp = jnp.zeros((NPAD, D_IN), jnp.float32).at[:N].set(x)

    deg_parts = _deg_kernel(dstp)
    dinv = _dinv_kernel(deg_parts)

    w_stack = jnp.stack([
        jnp.zeros((H, H), jnp.float32).at[:D_IN].set(W1),
        W2,
    ])
    b_stack = jnp.stack([b1.reshape(1, H), b2.reshape(1, H)])
    h0 = jnp.zeros((NPAD, H), jnp.float32).at[:, :D_IN].set(xp)

    def layer(h, wb):
        w, b = wb
        g = _pre_kernel(h, w, dinv)
        m = _prop_kernel(g, srcp, dstp)
        h_next = _post_kernel(m, g, dinv, b)
        return h_next, h_next

    _, hs = lax.scan(layer, h0, (w_stack, b_stack))

    out = _final_kernel(xp, hs[0], hs[1], Wlin, blin.reshape(1, D_OUT))
    return out[:N]
